# Initial kernel scaffold; baseline (speedup 1.0000x reference)
import jax
import jax.numpy as jnp
from jax.experimental import pallas as pl


def kernel(x_crime, x_regions, x_ext, s_crime, W_h, W_e, Wq, Wk):
    raise NotImplementedError("write your pallas kernel here")


# speedup vs baseline: 1.2363x; 1.2363x over previous
import functools
import math

import jax
import jax.numpy as jnp
from jax.experimental import pallas as pl

_TS = 20
_NHID = 32
_ATT_DOT = 32
_NFEAT = 16
_ALPHA = 0.2
_TARGET = 7
_SCALE = 1.0 / math.sqrt(_ATT_DOT)


def _gat_step(crime_ref, side_ref, ext_ref, wh_ref, we_ref, wq_ref, wk_ref,
              on_ref, en_ref):
    c = crime_ref[0].astype(jnp.float32)
    s = side_ref[0].astype(jnp.float32)
    e = ext_ref[:, 0].astype(jnp.float32)
    wh0 = wh_ref[0, 0:1, :]
    wh1 = wh_ref[0, 1:2, :]

    n, b = c.shape
    h3 = (c[:, :, None] * wh0[None, :, :]
          + s[:, :, None] * wh1[None, :, :])
    h2 = h3.reshape(n * b, _NHID)
    eh2 = jnp.dot(e.reshape(n * b, _NFEAT), we_ref[0],
                  preferred_element_type=jnp.float32)
    k2 = jnp.dot(h2, wk_ref[0], preferred_element_type=jnp.float32)
    q7 = jnp.dot(h3[_TARGET], wq_ref[0],
                 preferred_element_type=jnp.float32)

    k3 = k2.reshape(n, b, _ATT_DOT)
    logits = jnp.sum(q7[None, :, :] * k3, axis=2)
    logits = jnp.where(logits >= 0, logits, _ALPHA * logits) * _SCALE
    mx = jnp.max(logits, axis=0, keepdims=True)
    p = jnp.exp(logits - mx)
    attn = p / jnp.sum(p, axis=0, keepdims=True)

    on_ref[0] = jnp.sum(attn[:, :, None] * h3, axis=0)
    eh3 = eh2.reshape(n, b, _NHID)
    en_ref[0] = jnp.sum(attn[:, :, None] * eh3, axis=0)


@functools.partial(jax.jit, static_argnums=())
def kernel(x_crime, x_regions, x_ext, s_crime, W_h, W_e, Wq, Wk):
    B, T = x_crime.shape
    N = x_ext.shape[0]
    t0 = T - _TS

    regions = jnp.concatenate([x_regions, x_crime.T[None, :, :]], axis=0)
    crime_sl = regions[:, t0:, :].transpose(1, 0, 2)
    side_sl = s_crime[:, t0:, :].transpose(1, 0, 2)

    on_all, en_all = pl.pallas_call(
        _gat_step,
        grid=(_TS,),
        in_specs=[
            pl.BlockSpec((1, N, B), lambda j: (j, 0, 0)),
            pl.BlockSpec((1, N, B), lambda j: (j, 0, 0)),
            pl.BlockSpec((N, 1, B, _NFEAT), lambda j: (0, t0 + j, 0, 0)),
            pl.BlockSpec((1, 2, _NHID), lambda j: (j, 0, 0)),
            pl.BlockSpec((1, _NFEAT, _NHID), lambda j: (j, 0, 0)),
            pl.BlockSpec((1, _NHID, _ATT_DOT), lambda j: (j, 0, 0)),
            pl.BlockSpec((1, _NHID, _ATT_DOT), lambda j: (j, 0, 0)),
        ],
        out_specs=[
            pl.BlockSpec((1, B, _NHID), lambda j: (j, 0, 0)),
            pl.BlockSpec((1, B, _NHID), lambda j: (j, 0, 0)),
        ],
        out_shape=[
            jax.ShapeDtypeStruct((_TS, B, _NHID), jnp.float32),
            jax.ShapeDtypeStruct((_TS, B, _NHID), jnp.float32),
        ],
    )(crime_sl, side_sl, x_ext, W_h, W_e, Wq, Wk)

    return jnp.stack([on_all, en_all], axis=-1).transpose(1, 0, 2, 3)


# speedup vs baseline: 11.4974x; 9.2997x over previous
import math

import jax
import jax.numpy as jnp
from jax.experimental import pallas as pl

_TS = 20
_NHID = 32
_ATT_DOT = 32
_NFEAT = 16
_ALPHA = 0.2
_TARGET = 7
_SCALE = 1.0 / math.sqrt(_ATT_DOT)


def _gat_step(crime_ref, side_ref, ext_ref, wh_ref, we_ref, wq_ref, wk_ref,
              on_ref, en_ref):
    c = crime_ref[0].astype(jnp.float32)
    s = side_ref[0].astype(jnp.float32)
    wh = wh_ref[0]
    we = we_ref[0]

    qk = jax.lax.dot_general(wq_ref[0], wk_ref[0], (((1,), (1,)), ((), ())),
                             preferred_element_type=jnp.float32)
    b1 = jax.lax.dot_general(wh, qk, (((1,), (0,)), ((), ())),
                             preferred_element_type=jnp.float32)
    a2 = jax.lax.dot_general(b1, wh, (((1,), (1,)), ((), ())),
                             preferred_element_type=jnp.float32)

    c7 = c[_TARGET:_TARGET + 1, :]
    s7 = s[_TARGET:_TARGET + 1, :]
    raw = (c7 * (a2[0:1, 0:1] * c + a2[0:1, 1:2] * s)
           + s7 * (a2[1:2, 0:1] * c + a2[1:2, 1:2] * s))
    logits = jnp.where(raw >= 0, raw, raw * _ALPHA) * _SCALE

    mx = jnp.max(logits, axis=0, keepdims=True)
    p = jnp.exp(logits - mx)
    attn = p * (1.0 / jnp.sum(p, axis=0, keepdims=True))

    m_cs = jnp.concatenate(
        [jnp.sum(attn * c, axis=0, keepdims=True),
         jnp.sum(attn * s, axis=0, keepdims=True)], axis=0)
    m_e = jnp.concatenate(
        [jnp.sum(attn * ext_ref[0, :, f, :].astype(jnp.float32),
                 axis=0, keepdims=True) for f in range(_NFEAT)],
        axis=0)

    on_ref[0] = jax.lax.dot_general(wh, m_cs, (((0,), (0,)), ((), ())),
                                    preferred_element_type=jnp.float32)
    en_ref[0] = jax.lax.dot_general(we, m_e, (((0,), (0,)), ((), ())),
                                    preferred_element_type=jnp.float32)


def kernel(x_crime, x_regions, x_ext, s_crime, W_h, W_e, Wq, Wk):
    B, T = x_crime.shape
    N = x_ext.shape[0]
    t0 = T - _TS

    regions = jnp.concatenate([x_regions, x_crime.T[None, :, :]], axis=0)
    crime_sl = regions[:, t0:, :].transpose(1, 0, 2)
    side_sl = s_crime[:, t0:, :].transpose(1, 0, 2)
    ext_sl = x_ext[:, t0:, :, :].transpose(1, 0, 3, 2)

    on_all, en_all = pl.pallas_call(
        _gat_step,
        grid=(_TS,),
        in_specs=[
            pl.BlockSpec((1, N, B), lambda j: (j, 0, 0)),
            pl.BlockSpec((1, N, B), lambda j: (j, 0, 0)),
            pl.BlockSpec((1, N, _NFEAT, B), lambda j: (j, 0, 0, 0)),
            pl.BlockSpec((1, 2, _NHID), lambda j: (j, 0, 0)),
            pl.BlockSpec((1, _NFEAT, _NHID), lambda j: (j, 0, 0)),
            pl.BlockSpec((1, _NHID, _ATT_DOT), lambda j: (j, 0, 0)),
            pl.BlockSpec((1, _NHID, _ATT_DOT), lambda j: (j, 0, 0)),
        ],
        out_specs=[
            pl.BlockSpec((1, _NHID, B), lambda j: (j, 0, 0)),
            pl.BlockSpec((1, _NHID, B), lambda j: (j, 0, 0)),
        ],
        out_shape=[
            jax.ShapeDtypeStruct((_TS, _NHID, B), jnp.float32),
            jax.ShapeDtypeStruct((_TS, _NHID, B), jnp.float32),
        ],
    )(crime_sl, side_sl, ext_sl, W_h, W_e, Wq, Wk)

    return jnp.stack([on_all, en_all], axis=0).transpose(3, 1, 2, 0)


# speedup vs baseline: 13.3522x; 1.1613x over previous
import math

import jax
import jax.numpy as jnp
from jax.experimental import pallas as pl

_TS = 20
_NHID = 32
_ATT_DOT = 32
_NFEAT = 16
_ALPHA = 0.2
_TARGET = 7
_SCALE = 1.0 / math.sqrt(_ATT_DOT)


def _gat_step(crime_ref, side_ref, ext_ref, wh_ref, we_ref, wq_ref, wk_ref,
              on_ref, en_ref):
    c = crime_ref[0].astype(jnp.float32)
    s = side_ref[0].astype(jnp.float32)
    ef = ext_ref[0].astype(jnp.float32)
    wh = wh_ref[0]
    wh_t = wh.T
    we_t = we_ref[0].T

    u = jax.lax.dot_general(wh, wq_ref[0], (((1,), (0,)), ((), ())),
                            preferred_element_type=jnp.float32)
    v = jax.lax.dot_general(wh, wk_ref[0], (((1,), (0,)), ((), ())),
                            preferred_element_type=jnp.float32)
    a_c0 = jnp.sum(u * v[0:1, :], axis=1, keepdims=True)
    a_c1 = jnp.sum(u * v[1:2, :], axis=1, keepdims=True)

    c7 = c[_TARGET:_TARGET + 1, :]
    s7 = s[_TARGET:_TARGET + 1, :]
    raw = (c7 * (a_c0[0:1, :] * c + a_c1[0:1, :] * s)
           + s7 * (a_c0[1:2, :] * c + a_c1[1:2, :] * s))
    logits = jnp.where(raw >= 0, raw, raw * _ALPHA) * _SCALE

    mx = jnp.max(logits, axis=0, keepdims=True)
    p = jnp.exp(logits - mx)
    attn = p * (1.0 / jnp.sum(p, axis=0, keepdims=True))

    wc = jnp.sum(attn * c, axis=0, keepdims=True)
    ws = jnp.sum(attn * s, axis=0, keepdims=True)
    on_t = wh_t[:, 0:1] * wc + wh_t[:, 1:2] * ws
    en_t = jnp.zeros_like(on_t)
    for f in range(_NFEAT):
        g_f = jnp.sum(attn * ef[:, f, :], axis=0, keepdims=True)
        en_t = en_t + we_t[:, f:f + 1] * g_f
    on_ref[0] = on_t
    en_ref[0] = en_t


def kernel(x_crime, x_regions, x_ext, s_crime, W_h, W_e, Wq, Wk):
    B, T = x_crime.shape
    N = x_ext.shape[0]
    t0 = T - _TS

    regions = jnp.concatenate([x_regions, x_crime.T[None, :, :]], axis=0)
    crime_sl = regions[:, t0:, :].transpose(1, 0, 2)
    side_sl = s_crime[:, t0:, :].transpose(1, 0, 2)
    ext_sl = x_ext[:, t0:, :, :].transpose(1, 0, 3, 2)

    on_all, en_all = pl.pallas_call(
        _gat_step,
        grid=(_TS,),
        in_specs=[
            pl.BlockSpec((1, N, B), lambda j: (j, 0, 0)),
            pl.BlockSpec((1, N, B), lambda j: (j, 0, 0)),
            pl.BlockSpec((1, N, _NFEAT, B), lambda j: (j, 0, 0, 0)),
            pl.BlockSpec((1, 2, _NHID), lambda j: (j, 0, 0)),
            pl.BlockSpec((1, _NFEAT, _NHID), lambda j: (j, 0, 0)),
            pl.BlockSpec((1, _NHID, _ATT_DOT), lambda j: (j, 0, 0)),
            pl.BlockSpec((1, _NHID, _ATT_DOT), lambda j: (j, 0, 0)),
        ],
        out_specs=[
            pl.BlockSpec((1, _NHID, B), lambda j: (j, 0, 0)),
            pl.BlockSpec((1, _NHID, B), lambda j: (j, 0, 0)),
        ],
        out_shape=[
            jax.ShapeDtypeStruct((_TS, _NHID, B), jnp.float32),
            jax.ShapeDtypeStruct((_TS, _NHID, B), jnp.float32),
        ],
    )(crime_sl, side_sl, ext_sl, W_h, W_e, Wq, Wk)

    return jnp.stack([on_all, en_all], axis=0).transpose(3, 1, 2, 0)


# speedup vs baseline: 15.6220x; 1.1700x over previous
import math

import jax
import jax.numpy as jnp
from jax.experimental import pallas as pl

_TS = 20
_NHID = 32
_ATT_DOT = 32
_NFEAT = 16
_ALPHA = 0.2
_TARGET = 7
_SCALE = 1.0 / math.sqrt(_ATT_DOT)


def _gat_step(xr_ref, xc_ref, side_ref, ext_ref, wh_ref, we_ref, wq_ref,
              wk_ref, out_ref):
    c = jnp.concatenate([xr_ref[:, 0, 0, :], xc_ref[0]],
                        axis=0).astype(jnp.float32)
    s = side_ref[:, 0, 0, :].astype(jnp.float32)
    ef = ext_ref[0].astype(jnp.float32)
    wh = wh_ref[0]
    wh_t = wh.T
    we_t = we_ref[0].T

    u = jax.lax.dot_general(wh, wq_ref[0], (((1,), (0,)), ((), ())),
                            preferred_element_type=jnp.float32)
    v = jax.lax.dot_general(wh, wk_ref[0], (((1,), (0,)), ((), ())),
                            preferred_element_type=jnp.float32)
    a_c0 = jnp.sum(u * v[0:1, :], axis=1, keepdims=True)
    a_c1 = jnp.sum(u * v[1:2, :], axis=1, keepdims=True)

    c7 = c[_TARGET:_TARGET + 1, :]
    s7 = s[_TARGET:_TARGET + 1, :]
    raw = (c7 * (a_c0[0:1, :] * c + a_c1[0:1, :] * s)
           + s7 * (a_c0[1:2, :] * c + a_c1[1:2, :] * s))
    logits = jnp.where(raw >= 0, raw, raw * _ALPHA) * _SCALE

    mx = jnp.max(logits, axis=0, keepdims=True)
    p = jnp.exp(logits - mx)
    attn = p * (1.0 / jnp.sum(p, axis=0, keepdims=True))

    wc = jnp.sum(attn * c, axis=0, keepdims=True)
    ws = jnp.sum(attn * s, axis=0, keepdims=True)
    on_t = wh_t[:, 0:1] * wc + wh_t[:, 1:2] * ws
    en_t = jnp.zeros_like(on_t)
    for f in range(_NFEAT):
        g_f = jnp.sum(attn * ef[:, f, :], axis=0, keepdims=True)
        en_t = en_t + we_t[:, f:f + 1] * g_f
    out_ref[0, 0] = on_t
    out_ref[1, 0] = en_t


def kernel(x_crime, x_regions, x_ext, s_crime, W_h, W_e, Wq, Wk):
    B, T = x_crime.shape
    N = x_ext.shape[0]
    t0 = T - _TS

    xc_t = x_crime.T.reshape(T, 1, B)
    xr4 = x_regions.reshape(N - 1, T, 1, B)
    sc4 = s_crime.reshape(N, T, 1, B)
    ext_sl = x_ext[:, t0:, :, :].transpose(1, 0, 3, 2)

    out = pl.pallas_call(
        _gat_step,
        grid=(_TS,),
        in_specs=[
            pl.BlockSpec((N - 1, 1, 1, B), lambda j: (0, t0 + j, 0, 0)),
            pl.BlockSpec((1, 1, B), lambda j: (t0 + j, 0, 0)),
            pl.BlockSpec((N, 1, 1, B), lambda j: (0, t0 + j, 0, 0)),
            pl.BlockSpec((1, N, _NFEAT, B), lambda j: (j, 0, 0, 0)),
            pl.BlockSpec((1, 2, _NHID), lambda j: (j, 0, 0)),
            pl.BlockSpec((1, _NFEAT, _NHID), lambda j: (j, 0, 0)),
            pl.BlockSpec((1, _NHID, _ATT_DOT), lambda j: (j, 0, 0)),
            pl.BlockSpec((1, _NHID, _ATT_DOT), lambda j: (j, 0, 0)),
        ],
        out_specs=pl.BlockSpec((2, 1, _NHID, B), lambda j: (0, j, 0, 0)),
        out_shape=jax.ShapeDtypeStruct((2, _TS, _NHID, B), jnp.float32),
    )(xr4, xc_t, sc4, ext_sl, W_h, W_e, Wq, Wk)

    return out.transpose(3, 1, 2, 0)


# speedup vs baseline: 18.8133x; 1.2043x over previous
import math

import jax
import jax.numpy as jnp
from jax.experimental import pallas as pl

_TS = 20
_JPB = 2
_NHID = 32
_ATT_DOT = 32
_NFEAT = 16
_ALPHA = 0.2
_TARGET = 7
_SCALE = 1.0 / math.sqrt(_ATT_DOT)


def _one_step(c, s, ef, wh, we_t, wq, wk):
    wh_t = wh.T

    u = jax.lax.dot_general(wh, wq, (((1,), (0,)), ((), ())),
                            preferred_element_type=jnp.float32)
    v = jax.lax.dot_general(wh, wk, (((1,), (0,)), ((), ())),
                            preferred_element_type=jnp.float32)
    a_c0 = jnp.sum(u * v[0:1, :], axis=1, keepdims=True)
    a_c1 = jnp.sum(u * v[1:2, :], axis=1, keepdims=True)

    c7 = c[_TARGET:_TARGET + 1, :]
    s7 = s[_TARGET:_TARGET + 1, :]
    raw = (c7 * (a_c0[0:1, :] * c + a_c1[0:1, :] * s)
           + s7 * (a_c0[1:2, :] * c + a_c1[1:2, :] * s))
    logits = jnp.where(raw >= 0, raw, raw * _ALPHA) * _SCALE

    mx = jnp.max(logits, axis=0, keepdims=True)
    p = jnp.exp(logits - mx)
    attn = p * (1.0 / jnp.sum(p, axis=0, keepdims=True))

    wc = jnp.sum(attn * c, axis=0, keepdims=True)
    ws = jnp.sum(attn * s, axis=0, keepdims=True)
    on_t = wh_t[:, 0:1] * wc + wh_t[:, 1:2] * ws
    en_t = jnp.zeros_like(on_t)
    for f in range(_NFEAT):
        g_f = jnp.sum(attn * ef[:, f, :], axis=0, keepdims=True)
        en_t = en_t + we_t[:, f:f + 1] * g_f
    return on_t, en_t


def _gat_step(xr_ref, xc_ref, side_ref, ext_ref, wh_ref, we_ref, wq_ref,
              wk_ref, out_ref):
    for k in range(_JPB):
        c = jnp.concatenate([xr_ref[:, k, 0, :], xc_ref[k]],
                            axis=0).astype(jnp.float32)
        s = side_ref[:, k, 0, :].astype(jnp.float32)
        ef = ext_ref[k].astype(jnp.float32)
        on_t, en_t = _one_step(c, s, ef, wh_ref[k], we_ref[k].T,
                               wq_ref[k], wk_ref[k])
        out_ref[0, k] = on_t
        out_ref[1, k] = en_t


def kernel(x_crime, x_regions, x_ext, s_crime, W_h, W_e, Wq, Wk):
    B, T = x_crime.shape
    N = x_ext.shape[0]
    t0 = T - _TS
    nsteps = _TS // _JPB

    xc_t = x_crime.T.reshape(T, 1, B)
    xr4 = x_regions.reshape(N - 1, T, 1, B)
    sc4 = s_crime.reshape(N, T, 1, B)
    ext_sl = x_ext[:, t0:, :, :].transpose(1, 0, 3, 2)

    out = pl.pallas_call(
        _gat_step,
        grid=(nsteps,),
        in_specs=[
            pl.BlockSpec((N - 1, _JPB, 1, B),
                         lambda j: (0, t0 // _JPB + j, 0, 0)),
            pl.BlockSpec((_JPB, 1, B), lambda j: (t0 // _JPB + j, 0, 0)),
            pl.BlockSpec((N, _JPB, 1, B),
                         lambda j: (0, t0 // _JPB + j, 0, 0)),
            pl.BlockSpec((_JPB, N, _NFEAT, B), lambda j: (j, 0, 0, 0)),
            pl.BlockSpec((_JPB, 2, _NHID), lambda j: (j, 0, 0)),
            pl.BlockSpec((_JPB, _NFEAT, _NHID), lambda j: (j, 0, 0)),
            pl.BlockSpec((_JPB, _NHID, _ATT_DOT), lambda j: (j, 0, 0)),
            pl.BlockSpec((_JPB, _NHID, _ATT_DOT), lambda j: (j, 0, 0)),
        ],
        out_specs=pl.BlockSpec((2, _JPB, _NHID, B), lambda j: (0, j, 0, 0)),
        out_shape=jax.ShapeDtypeStruct((2, _TS, _NHID, B), jnp.float32),
    )(xr4, xc_t, sc4, ext_sl, W_h, W_e, Wq, Wk)

    return out.transpose(3, 1, 2, 0)


# speedup vs baseline: 20.0670x; 1.0666x over previous
import math

import jax
import jax.numpy as jnp
from jax.experimental import pallas as pl

_TS = 20
_JPB = 4
_NHID = 32
_ATT_DOT = 32
_NFEAT = 16
_ALPHA = 0.2
_TARGET = 7
_SCALE = 1.0 / math.sqrt(_ATT_DOT)


def _one_step(c, s, ef, wh, we_t, wq, wk):
    wh_t = wh.T

    u = jax.lax.dot_general(wh, wq, (((1,), (0,)), ((), ())),
                            preferred_element_type=jnp.float32)
    v = jax.lax.dot_general(wh, wk, (((1,), (0,)), ((), ())),
                            preferred_element_type=jnp.float32)
    a_c0 = jnp.sum(u * v[0:1, :], axis=1, keepdims=True)
    a_c1 = jnp.sum(u * v[1:2, :], axis=1, keepdims=True)

    c7 = c[_TARGET:_TARGET + 1, :]
    s7 = s[_TARGET:_TARGET + 1, :]
    raw = (c7 * (a_c0[0:1, :] * c + a_c1[0:1, :] * s)
           + s7 * (a_c0[1:2, :] * c + a_c1[1:2, :] * s))
    logits = jnp.where(raw >= 0, raw, raw * _ALPHA) * _SCALE

    mx = jnp.max(logits, axis=0, keepdims=True)
    p = jnp.exp(logits - mx)
    attn = p * (1.0 / jnp.sum(p, axis=0, keepdims=True))

    wc = jnp.sum(attn * c, axis=0, keepdims=True)
    ws = jnp.sum(attn * s, axis=0, keepdims=True)
    on_t = wh_t[:, 0:1] * wc + wh_t[:, 1:2] * ws
    en_t = jnp.zeros_like(on_t)
    for f in range(_NFEAT):
        g_f = jnp.sum(attn * ef[:, f, :], axis=0, keepdims=True)
        en_t = en_t + we_t[:, f:f + 1] * g_f
    return on_t, en_t


def _gat_step(xr_ref, xc_ref, side_ref, ext_ref, wh_ref, we_ref, wq_ref,
              wk_ref, out_ref):
    for k in range(_JPB):
        c = jnp.concatenate([xr_ref[:, k, 0, :], xc_ref[k]],
                            axis=0).astype(jnp.float32)
        s = side_ref[:, k, 0, :].astype(jnp.float32)
        ef = ext_ref[k].astype(jnp.float32)
        on_t, en_t = _one_step(c, s, ef, wh_ref[k], we_ref[k].T,
                               wq_ref[k], wk_ref[k])
        out_ref[0, k] = on_t
        out_ref[1, k] = en_t


def kernel(x_crime, x_regions, x_ext, s_crime, W_h, W_e, Wq, Wk):
    B, T = x_crime.shape
    N = x_ext.shape[0]
    t0 = T - _TS
    nsteps = _TS // _JPB

    xc_t = x_crime.T.reshape(T, 1, B)
    xr4 = x_regions.reshape(N - 1, T, 1, B)
    sc4 = s_crime.reshape(N, T, 1, B)
    ext_sl = x_ext[:, t0:, :, :].transpose(1, 0, 3, 2)

    out = pl.pallas_call(
        _gat_step,
        grid=(nsteps,),
        in_specs=[
            pl.BlockSpec((N - 1, _JPB, 1, B),
                         lambda j: (0, t0 // _JPB + j, 0, 0)),
            pl.BlockSpec((_JPB, 1, B), lambda j: (t0 // _JPB + j, 0, 0)),
            pl.BlockSpec((N, _JPB, 1, B),
                         lambda j: (0, t0 // _JPB + j, 0, 0)),
            pl.BlockSpec((_JPB, N, _NFEAT, B), lambda j: (j, 0, 0, 0)),
            pl.BlockSpec((_JPB, 2, _NHID), lambda j: (j, 0, 0)),
            pl.BlockSpec((_JPB, _NFEAT, _NHID), lambda j: (j, 0, 0)),
            pl.BlockSpec((_JPB, _NHID, _ATT_DOT), lambda j: (j, 0, 0)),
            pl.BlockSpec((_JPB, _NHID, _ATT_DOT), lambda j: (j, 0, 0)),
        ],
        out_specs=pl.BlockSpec((2, _JPB, _NHID, B), lambda j: (0, j, 0, 0)),
        out_shape=jax.ShapeDtypeStruct((2, _TS, _NHID, B), jnp.float32),
    )(xr4, xc_t, sc4, ext_sl, W_h, W_e, Wq, Wk)

    return out.transpose(3, 1, 2, 0)
